# baseline (speedup 1.0000x reference)
import jax
import jax.numpy as jnp
from jax import lax
from jax.experimental import pallas as pl
from jax.experimental.pallas import tpu as pltpu
from jax.experimental.pallas import tpu_sc as plsc

RANK = 32
BATCH = 16384
NC = 2
NS = 16
NW = NC * NS
B_PER_W = BATCH // NW
N_GROUPS = B_PER_W // 16
IDX_CHUNK = 128
N_CHUNKS = B_PER_W // IDX_CHUNK


def _factorizer_kernel(inputs_hbm, ie_hbm, je_hbm, ib_hbm, jb_hbm, const_hbm,
                       out_hbm,
                       idx2_v, idx_i_v, idx_j_v, ie_rows, je_rows,
                       ib_v, jb_v, const_v, tr_v, out_v, sem):
  wid = lax.axis_index("s") * NC + lax.axis_index("c")
  base = wid * B_PER_W

  pltpu.sync_copy(inputs_hbm.at[pl.ds(base * 2, B_PER_W * 2)], idx2_v)
  pltpu.sync_copy(const_hbm, const_v)

  lane = lax.iota(jnp.int32, 16)
  for k in range(B_PER_W // 16):
    pos = jnp.full((16,), k * 32, jnp.int32) + lane * 2
    vi = plsc.load_gather(idx2_v, [pos])
    vj = plsc.load_gather(idx2_v, [pos + 1])
    r, c0 = divmod(k * 16, IDX_CHUNK)
    idx_i_v[r, pl.ds(c0, 16)] = vi
    idx_j_v[r, pl.ds(c0, 16)] = vj

  copies = []
  for q in range(N_CHUNKS):
    sl = pl.ds(q * IDX_CHUNK, IDX_CHUNK)
    copies.append(pltpu.async_copy(ie_hbm.at[idx_i_v.at[q]], ie_rows.at[sl], sem))
    copies.append(pltpu.async_copy(je_hbm.at[idx_j_v.at[q]], je_rows.at[sl], sem))
    copies.append(pltpu.async_copy(ib_hbm.at[idx_i_v.at[q]], ib_v.at[sl], sem))
    copies.append(pltpu.async_copy(jb_hbm.at[idx_j_v.at[q]], jb_v.at[sl], sem))
  for cp in copies:
    cp.wait()

  cval = const_v[...]

  def group_body(g, carry):
    b0 = g * 16
    for e in range(16):
      b = b0 + e
      p = (ie_rows[b, pl.ds(0, 16)] * je_rows[b, pl.ds(0, 16)]
           + ie_rows[b, pl.ds(16, 16)] * je_rows[b, pl.ds(16, 16)])
      tr_v[pl.ds(e * 24, 16)] = p
    acc = jnp.zeros((16,), jnp.float32)
    for c in range(16):
      col = plsc.load_gather(tr_v, [lane * 24 + c])
      acc = acc + col
    out_v[pl.ds(b0, 16)] = acc + ib_v[pl.ds(b0, 16)] + jb_v[pl.ds(b0, 16)] + cval
    return carry

  lax.fori_loop(0, N_GROUPS, group_body, 0)

  pltpu.sync_copy(out_v, out_hbm.at[pl.ds(base, B_PER_W)])


@jax.jit
def _run(inputs, i_embedding, j_embedding, i_bias_flat, j_bias_flat, constant):
  mesh = plsc.VectorSubcoreMesh(core_axis_name="c", subcore_axis_name="s")
  fn = pl.kernel(
      _factorizer_kernel,
      out_type=jax.ShapeDtypeStruct((BATCH,), jnp.float32),
      mesh=mesh,
      compiler_params=pltpu.CompilerParams(
          needs_layout_passes=False, use_tc_tiling_on_sc=False),
      scratch_types=[
          pltpu.VMEM((B_PER_W * 2,), jnp.int32),
          pltpu.VMEM((N_CHUNKS, IDX_CHUNK), jnp.int32),
          pltpu.VMEM((N_CHUNKS, IDX_CHUNK), jnp.int32),
          pltpu.VMEM((B_PER_W, RANK), jnp.float32),
          pltpu.VMEM((B_PER_W, RANK), jnp.float32),
          pltpu.VMEM((B_PER_W,), jnp.float32),
          pltpu.VMEM((B_PER_W,), jnp.float32),
          pltpu.VMEM((16,), jnp.float32),
          pltpu.VMEM((16 * 24,), jnp.float32),
          pltpu.VMEM((B_PER_W,), jnp.float32),
          pltpu.SemaphoreType.DMA,
      ],
  )
  return fn(inputs, i_embedding, j_embedding, i_bias_flat, j_bias_flat,
            constant)


def kernel(inputs, i_embedding, j_embedding, i_bias, j_bias, constant):
  out = _run(inputs.astype(jnp.int32).reshape(-1), i_embedding, j_embedding,
             i_bias.reshape(-1), j_bias.reshape(-1),
             jnp.broadcast_to(constant.reshape(-1), (16,)))
  return out.reshape(BATCH, 1)
